# baseline (speedup 1.0000x reference)
import functools

import jax
import jax.numpy as jnp
from jax.experimental import pallas as pl
from jax.experimental.pallas import tpu as pltpu


def _pred_masks(no_b, tgt_b):
    n0 = no_b[0:1, :]
    n1 = no_b[1:2, :]
    n2 = no_b[2:3, :]
    p0 = (n0 >= n1) & (n0 >= n2)
    p1 = jnp.logical_not(p0) & (n1 >= n2)
    p2 = jnp.logical_not(p0 | p1)
    km = ((tgt_b == 1) & p1).astype(jnp.float32)
    tm = ((tgt_b == 2) & p2).astype(jnp.float32)
    tw = ((tgt_b == 2) & jnp.logical_not(p2)).astype(jnp.float32)
    return km, tm, tw


def _fused_kernel(no_ref, tg_ref, f_ref, dq_ref, out_ref,
                  vec_ref, cnt_ref, proto_ref, cols_ref, acc_ref,
                  *, batch, n_total, q):
    ph = pl.program_id(0)
    i = pl.program_id(1)
    nblocks = pl.num_programs(1)
    inv_n = 1.0 / n_total

    @pl.when((ph == 0) & (i == 0))
    def _init():
        vec_ref[...] = jnp.zeros_like(vec_ref)
        for b in range(batch):
            for j in range(3):
                cnt_ref[b, j] = 0.0

    @pl.when(ph == 0)
    def _pass1():
        for b in range(batch):
            km, tm, tw = _pred_masks(no_ref[b], tg_ref[b])
            f = f_ref[b]
            vec_ref[:, b:b + 1] += jnp.sum(f * km, axis=1, keepdims=True)
            vec_ref[:, batch + b:batch + b + 1] += (
                jnp.sum(f * tm, axis=1, keepdims=True))
            cnt_ref[b, 0] += jnp.sum(tm)
            cnt_ref[b, 1] += jnp.sum(tw)
            cnt_ref[b, 2] += jnp.sum(km)

    @pl.when((ph == 1) & (i == 0))
    def _prologue():
        cols = vec_ref[...] * inv_n
        cnorm = jnp.sqrt(jnp.sum(cols * cols, axis=0, keepdims=True)) + 1e-8
        cols_ref[...] = cols / cnorm
        dq = dq_ref[...]
        dnorm = jnp.sqrt(jnp.sum(dq * dq, axis=1, keepdims=True)) + 1e-8
        proto_ref[...] = dq / dnorm
        for j in range(2 * batch):
            acc_ref[j] = 0.0

    @pl.when(ph == 1)
    def _pass2():
        ka0 = cnt_ref[0, 2] > 0.0
        ka1 = cnt_ref[1, 2] > 0.0
        for b in range(batch):
            _, _, tw = _pred_masks(no_ref[b], tg_ref[b])
            f = f_ref[b]
            sq = jnp.sum(f * f, axis=0, keepdims=True)
            rn = 1.0 / (jnp.sqrt(sq) + 1e-8)
            dots = jnp.dot(proto_ref[...], f,
                           preferred_element_type=jnp.float32)
            ek = jnp.sum(jnp.exp(dots * rn) * tw)
            ckv = jax.lax.dot_general(
                cols_ref[:, 0:batch], f, (((0,), (0,)), ((), ())),
                preferred_element_type=jnp.float32)
            ekv = jnp.exp(ckv * rn) * tw
            ek = ek + jnp.where(ka0, jnp.sum(ekv[0:1, :]), 0.0)
            if b >= 1:
                ek = ek + jnp.where(ka1, jnp.sum(ekv[1:2, :]), 0.0)
            sv = jnp.sum(f * cols_ref[:, batch + b:batch + b + 1],
                         axis=0, keepdims=True)
            s_b = jnp.sum(sv * rn * tw)
            acc_ref[b] += s_b
            acc_ref[batch + b] += ek

    @pl.when((ph == 1) & (i == nblocks - 1))
    def _epilogue():
        et = jnp.float32(0.0)
        ek = jnp.float32(0.0)
        any_c = False
        for b in range(batch):
            c_b = (cnt_ref[b, 0] > 0.0) & (cnt_ref[b, 1] > 0.0)
            et = et + jnp.where(c_b, jnp.exp(acc_ref[b]), 0.0)
            ek = ek + jnp.where(c_b, acc_ref[batch + b], 0.0)
            any_c = c_b | any_c
        denom = jnp.where(any_c, ek, 1.0)
        loss = jnp.where(any_c, (-1.0 / batch) * jnp.log(et / denom), 0.0)
        out_ref[0, 0] = loss


@jax.jit
def _run(net_output, feature, target, kidney_deque):
    b, c, d, h, w = net_output.shape
    fd = feature.shape[1]
    q = kidney_deque.shape[0]
    n_total = d * h * w
    nb = 8192
    while n_total % nb != 0:
        nb //= 2
    nblocks = n_total // nb

    no = net_output.reshape(b, c, n_total)
    f = feature.reshape(b, fd, n_total)
    tg = target.reshape(b, 1, n_total)

    loss = pl.pallas_call(
        functools.partial(_fused_kernel, batch=b, n_total=n_total, q=q),
        grid=(2, nblocks),
        in_specs=[
            pl.BlockSpec((b, c, nb), lambda p, i: (0, 0, i)),
            pl.BlockSpec((b, 1, nb), lambda p, i: (0, 0, i)),
            pl.BlockSpec((b, fd, nb), lambda p, i: (0, 0, i)),
            pl.BlockSpec((q, fd), lambda p, i: (0, 0)),
        ],
        out_specs=pl.BlockSpec(memory_space=pltpu.SMEM),
        out_shape=jax.ShapeDtypeStruct((1, 1), jnp.float32),
        scratch_shapes=[
            pltpu.VMEM((fd, 2 * b * 2), jnp.float32),
            pltpu.SMEM((b, 3), jnp.float32),
            pltpu.VMEM((q, fd), jnp.float32),
            pltpu.VMEM((fd, 2 * b * 2), jnp.float32),
            pltpu.SMEM((2 * b,), jnp.float32),
        ],
    )(no, tg, f, kidney_deque)

    return loss[0, 0]


def kernel(net_output, feature, target, kidney_deque, background_deque):
    del background_deque
    return _run(net_output, feature, target, kidney_deque)


# speedup vs baseline: 1.0421x; 1.0421x over previous
import functools

import jax
import jax.numpy as jnp
from jax.experimental import pallas as pl
from jax.experimental.pallas import tpu as pltpu


def _pred_masks(no_b, tgt_b):
    n0 = no_b[0:1, :]
    n1 = no_b[1:2, :]
    n2 = no_b[2:3, :]
    p0 = (n0 >= n1) & (n0 >= n2)
    p1 = jnp.logical_not(p0) & (n1 >= n2)
    p2 = jnp.logical_not(p0 | p1)
    km = ((tgt_b == 1) & p1).astype(jnp.float32)
    tm = ((tgt_b == 2) & p2).astype(jnp.float32)
    tw = ((tgt_b == 2) & jnp.logical_not(p2)).astype(jnp.float32)
    return km, tm, tw


def _fused_kernel(no_ref, tg_ref, f_ref, dq_ref, out_ref,
                  vec_ref, cnt_ref, proto_ref, cols_ref, acc_ref,
                  *, batch, n_total, q):
    ph = pl.program_id(0)
    i = pl.program_id(1)
    nblocks = pl.num_programs(1)
    inv_n = 1.0 / n_total

    @pl.when((ph == 0) & (i == 0))
    def _init():
        vec_ref[...] = jnp.zeros_like(vec_ref)
        for b in range(batch):
            for j in range(3):
                cnt_ref[b, j] = 0.0

    @pl.when(ph == 0)
    def _pass1():
        for b in range(batch):
            km, tm, tw = _pred_masks(no_ref[b], tg_ref[b])
            f = f_ref[b]
            vec_ref[:, b:b + 1] += jnp.sum(f * km, axis=1, keepdims=True)
            vec_ref[:, batch + b:batch + b + 1] += (
                jnp.sum(f * tm, axis=1, keepdims=True))
            cnt_ref[b, 0] += jnp.sum(tm)
            cnt_ref[b, 1] += jnp.sum(tw)
            cnt_ref[b, 2] += jnp.sum(km)

    @pl.when((ph == 1) & (i == 0))
    def _prologue():
        cols = vec_ref[...] * inv_n
        cnorm = jnp.sqrt(jnp.sum(cols * cols, axis=0, keepdims=True)) + 1e-8
        cols_ref[...] = cols / cnorm
        dq = dq_ref[...]
        dnorm = jnp.sqrt(jnp.sum(dq * dq, axis=1, keepdims=True)) + 1e-8
        proto_ref[...] = dq / dnorm
        for j in range(2 * batch):
            acc_ref[j] = 0.0

    @pl.when(ph == 1)
    def _pass2():
        ka0 = cnt_ref[0, 2] > 0.0
        ka1 = cnt_ref[1, 2] > 0.0
        for b in range(batch):
            _, _, tw = _pred_masks(no_ref[b], tg_ref[b])
            f = f_ref[b]
            sq = jnp.sum(f * f, axis=0, keepdims=True)
            rn = 1.0 / (jnp.sqrt(sq) + 1e-8)
            dots = jnp.dot(proto_ref[...], f,
                           preferred_element_type=jnp.float32)
            ek = jnp.sum(jnp.exp(dots * rn) * tw)
            ckv = jax.lax.dot_general(
                cols_ref[:, 0:batch], f, (((0,), (0,)), ((), ())),
                preferred_element_type=jnp.float32)
            ekv = jnp.exp(ckv * rn) * tw
            ek = ek + jnp.where(ka0, jnp.sum(ekv[0:1, :]), 0.0)
            if b >= 1:
                ek = ek + jnp.where(ka1, jnp.sum(ekv[1:2, :]), 0.0)
            sv = jnp.sum(f * cols_ref[:, batch + b:batch + b + 1],
                         axis=0, keepdims=True)
            s_b = jnp.sum(sv * rn * tw)
            acc_ref[b] += s_b
            acc_ref[batch + b] += ek

    @pl.when((ph == 1) & (i == nblocks - 1))
    def _epilogue():
        et = jnp.float32(0.0)
        ek = jnp.float32(0.0)
        any_c = False
        for b in range(batch):
            c_b = (cnt_ref[b, 0] > 0.0) & (cnt_ref[b, 1] > 0.0)
            et = et + jnp.where(c_b, jnp.exp(acc_ref[b]), 0.0)
            ek = ek + jnp.where(c_b, acc_ref[batch + b], 0.0)
            any_c = c_b | any_c
        denom = jnp.where(any_c, ek, 1.0)
        loss = jnp.where(any_c, (-1.0 / batch) * jnp.log(et / denom), 0.0)
        out_ref[0, 0] = loss


@jax.jit
def _run(net_output, feature, target, kidney_deque):
    b, c, d, h, w = net_output.shape
    fd = feature.shape[1]
    q = kidney_deque.shape[0]
    n_total = d * h * w
    nb = 16384
    while n_total % nb != 0:
        nb //= 2
    nblocks = n_total // nb

    no = net_output.reshape(b, c, n_total)
    f = feature.reshape(b, fd, n_total)
    tg = target.reshape(b, 1, n_total)

    loss = pl.pallas_call(
        functools.partial(_fused_kernel, batch=b, n_total=n_total, q=q),
        grid=(2, nblocks),
        in_specs=[
            pl.BlockSpec((b, c, nb), lambda p, i: (0, 0, i)),
            pl.BlockSpec((b, 1, nb), lambda p, i: (0, 0, i)),
            pl.BlockSpec((b, fd, nb), lambda p, i: (0, 0, i)),
            pl.BlockSpec((q, fd), lambda p, i: (0, 0)),
        ],
        out_specs=pl.BlockSpec(memory_space=pltpu.SMEM),
        out_shape=jax.ShapeDtypeStruct((1, 1), jnp.float32),
        scratch_shapes=[
            pltpu.VMEM((fd, 2 * b * 2), jnp.float32),
            pltpu.SMEM((b, 3), jnp.float32),
            pltpu.VMEM((q, fd), jnp.float32),
            pltpu.VMEM((fd, 2 * b * 2), jnp.float32),
            pltpu.SMEM((2 * b,), jnp.float32),
        ],
    )(no, tg, f, kidney_deque)

    return loss[0, 0]


def kernel(net_output, feature, target, kidney_deque, background_deque):
    del background_deque
    return _run(net_output, feature, target, kidney_deque)
